# baseline (speedup 1.0000x reference)
import functools

import jax
import jax.numpy as jnp
from jax import lax
from jax.experimental import pallas as pl
from jax.experimental.pallas import tpu as pltpu
from jax.experimental.pallas import tpu_sc as plsc

_R = 512


@functools.lru_cache(maxsize=None)
def _make_relayout(V, D):
    grid = -(-V // (2 * _R))
    H2 = grid * _R

    def relayout_body(lo_ref, hi_ref, eye_ref, out_ref):
        del eye_ref
        out_ref[:, 0:D] = jnp.transpose(lo_ref[...])
        out_ref[:, D : 2 * D] = jnp.transpose(hi_ref[...])

    return pl.pallas_call(
        relayout_body,
        grid=(grid,),
        in_specs=[
            pl.BlockSpec((D, _R), lambda p: (0, 2 * p)),
            pl.BlockSpec((D, _R), lambda p: (0, 2 * p + 1)),
            pl.BlockSpec((D, D), lambda p: (0, 0)),
        ],
        out_specs=pl.BlockSpec((_R, 2 * D), lambda p: (p, 0)),
        out_shape=jax.ShapeDtypeStruct((H2, 2 * D), jnp.float32),
    )


@functools.lru_cache(maxsize=None)
def _make_gather(V, D, B, H2):
    info = plsc.get_sparse_core_info()
    NC, NS = info.num_cores, info.num_subcores
    NW = NC * NS
    assert B % (8 * NW) == 0
    DP = 2 * D
    b_per_w = B // NW
    mesh = plsc.VectorSubcoreMesh(core_axis_name="c", subcore_axis_name="s")

    @functools.partial(
        pl.kernel,
        mesh=mesh,
        compiler_params=pltpu.CompilerParams(
            use_tc_tiling_on_sc=True, needs_layout_passes=False
        ),
        out_type=jax.ShapeDtypeStruct((B, DP), jnp.float32),
        scratch_types=[
            pltpu.VMEM((b_per_w,), jnp.int32),
            pltpu.VMEM((b_per_w,), jnp.int32),
            pltpu.VMEM((b_per_w, DP), jnp.float32),
            pltpu.SemaphoreType.DMA,
        ],
    )
    def gather_kernel(idx_hbm, table_hbm, out_hbm, idx_v, bidx_v, rows_v, sem):
        wid = lax.axis_index("s") * NC + lax.axis_index("c")
        base = wid * b_per_w
        pltpu.sync_copy(idx_hbm.at[pl.ds(base, b_per_w)], idx_v)
        for g in range(b_per_w // 16):
            v = idx_v[pl.ds(g * 16, 16)]
            bidx_v[pl.ds(g * 16, 16)] = (
                lax.shift_left(lax.shift_right_logical(v, 10), 9)
                + (v & (_R - 1))
            )
        pltpu.async_copy(table_hbm.at[bidx_v], rows_v, sem).wait()

        def sel_group(g, carry):
            rpos = lax.iota(jnp.int32, 16) + g * 16
            v = idx_v[pl.ds(g * 16, 16)]
            half = (lax.shift_right_logical(v, 9) & 1) * D

            def sel_col(c, carry2):
                vec = plsc.load_gather(rows_v, [rpos, half + c])
                plsc.store_scatter(
                    rows_v, [rpos, jnp.full((16,), 0, jnp.int32) + c], vec
                )
                return carry2

            lax.fori_loop(0, D, sel_col, 0)
            return carry

        lax.fori_loop(0, b_per_w // 16, sel_group, 0)
        pltpu.sync_copy(rows_v, out_hbm.at[pl.ds(base, b_per_w)])

    return gather_kernel


def kernel(beta, emb_weight):
    (B,) = beta.shape
    V, D = emb_weight.shape
    beta = beta.astype(jnp.int32)
    eye = jnp.eye(D, dtype=jnp.float32)
    wt = emb_weight.T
    packed = _make_relayout(V, D)(wt, wt, eye)
    outp = _make_gather(V, D, B, packed.shape[0])(beta, packed)
    return outp[:, :D]
